# baseline (speedup 1.0000x reference)
import functools

import jax
import jax.numpy as jnp
from jax import lax
from jax.experimental import pallas as pl
from jax.experimental.pallas import tpu as pltpu
from jax.experimental.pallas import tpu_sc as plsc

NC = 2
NS = 16
LANES = 16
B = 64


def _sc_message_pass(N, D, E):
    nblk = E // B
    assert nblk * B == E
    nworkers = NC * NS
    nfull = nblk // nworkers
    nextra = nblk % nworkers
    ch = 80
    nch = N // ch
    assert nch * ch == N
    rounds = -(-nch // NS)

    mesh = plsc.VectorSubcoreMesh(core_axis_name="c", subcore_axis_name="s")

    assert nfull % 2 == 0

    @functools.partial(
        pl.kernel,
        mesh=mesh,
        out_type=jax.ShapeDtypeStruct((NC, N, D), jnp.float32),
        scratch_types=[
            pltpu.VMEM((B,), jnp.int32),
            pltpu.VMEM((B,), jnp.int32),
            pltpu.VMEM((B,), jnp.int32),
            pltpu.VMEM((B,), jnp.int32),
            pltpu.VMEM((B, D), jnp.float32),
            pltpu.VMEM((B, D), jnp.float32),
            pltpu.VMEM((B, D), jnp.float32),
            pltpu.VMEM((B, D), jnp.float32),
            pltpu.VMEM_SHARED((N, D), jnp.float32),
            pltpu.SemaphoreType.DMA,
            pltpu.SemaphoreType.DMA,
            pltpu.SemaphoreType.DMA,
            pltpu.SemaphoreType.DMA,
            pltpu.SemaphoreType.DMA,
            pltpu.SemaphoreType.DMA,
        ],
    )
    def k(node_hbm, src_hbm, dst_hbm, edge_hbm, part_hbm,
          si0, si1, di0, di1, m0, m1, g0, g1, acc,
          es0, es1, gs0, gs1, ss0, ss1):
        src_idx, dst_idx = (si0, si1), (di0, di1)
        m, g = (m0, m1), (g0, g1)
        esem, gsem, ssem = (es0, es1), (gs0, gs1), (ss0, ss1)
        cid = lax.axis_index("c")
        sid = lax.axis_index("s")
        wid = sid * NC + cid

        def zrow(r, _):
            for c in range(D // LANES):
                m0[r, pl.ds(c * LANES, LANES)] = jnp.zeros((LANES,), jnp.float32)
            return 0
        lax.fori_loop(0, B, zrow, 0)
        for kk in range(rounds):
            j = kk * NS + sid
            @pl.when(j < nch)
            def _():
                pltpu.sync_copy(m0.at[pl.ds(0, ch)], acc.at[pl.ds(j * ch, ch)])
        plsc.subcore_barrier()

        def issue(blk, b):
            off = blk * B
            pltpu.sync_copy(src_hbm.at[pl.ds(off, B)], src_idx[b])
            pltpu.sync_copy(dst_hbm.at[pl.ds(off, B)], dst_idx[b])
            pltpu.async_copy(edge_hbm.at[pl.ds(off, B)], m[b], esem[b])

        def wait_data(b):
            pltpu.make_async_copy(edge_hbm.at[pl.ds(0, B)], m[b], esem[b]).wait()
            pltpu.async_copy(node_hbm.at[src_idx[b]], g[b], gsem[b]).wait()

        def wait_scatter(b):
            pltpu.make_async_copy(m[b], acc.at[dst_idx[b]], ssem[b]).wait()

        def compute(b):
            mb, gb = m[b], g[b]

            def row(r, _):
                for c in range(D // LANES):
                    sl = pl.ds(c * LANES, LANES)
                    mb[r, sl] = jnp.maximum(mb[r, sl] + gb[r, sl], 0.0)
                return 0
            lax.fori_loop(0, B, row, 0)

        first = wid * nfull
        issue(first, 0)

        def body(i2, _):
            for b in (0, 1):
                i = i2 * 2 + b
                blk = first + i
                q = 1 - b
                wait_data(b)
                @pl.when(i + 1 < nfull)
                def _():
                    issue(blk + 1, q)
                compute(b)
                pltpu.sync_copy(m[b], acc.at[dst_idx[b]], add=True)
            return 0
        lax.fori_loop(0, nfull // 2, body, 0)

        if nextra:
            @pl.when(wid < nextra)
            def _():
                blk = nworkers * nfull + wid
                issue(blk, 0)
                wait_data(0)
                compute(0)
                pltpu.sync_copy(m0, acc.at[di0], add=True)

        plsc.subcore_barrier()
        for kk in range(rounds):
            j = kk * NS + sid
            @pl.when(j < nch)
            def _():
                r0 = j * ch
                pltpu.sync_copy(acc.at[pl.ds(r0, ch)], m0.at[pl.ds(0, ch)])
                pltpu.sync_copy(m0.at[pl.ds(0, ch)],
                                part_hbm.at[cid, pl.ds(r0, ch)])

    return k


def _combine(x_ref, p_ref, o_ref):
    o_ref[...] = x_ref[...] + p_ref[0] + p_ref[1]


def kernel(node_feat, edge_index, edge_feat):
    N, D = node_feat.shape
    E = edge_feat.shape[0]
    src = edge_index[0]
    dst = edge_index[1]
    parts = _sc_message_pass(N, D, E)(node_feat, src, dst, edge_feat)

    rb = 1000 if N % 1000 == 0 else N
    out = pl.pallas_call(
        _combine,
        grid=(N // rb,),
        in_specs=[
            pl.BlockSpec((rb, D), lambda i: (i, 0)),
            pl.BlockSpec((NC, rb, D), lambda i: (0, i, 0)),
        ],
        out_specs=pl.BlockSpec((rb, D), lambda i: (i, 0)),
        out_shape=jax.ShapeDtypeStruct((N, D), jnp.float32),
    )(node_feat, parts)
    return out


# speedup vs baseline: 1.8833x; 1.8833x over previous
import functools

import jax
import jax.numpy as jnp
from jax import lax
from jax.experimental import pallas as pl
from jax.experimental.pallas import tpu as pltpu
from jax.experimental.pallas import tpu_sc as plsc

NC = 2
NS = 16
LANES = 16
B = 128


def _sc_message_pass(N, D, E):
    nblk = E // B
    assert nblk * B == E
    nworkers = NC * NS
    nfull = nblk // nworkers
    nextra = nblk % nworkers
    ch = 80
    nch = N // ch
    assert nch * ch == N
    rounds = -(-nch // NS)

    mesh = plsc.VectorSubcoreMesh(core_axis_name="c", subcore_axis_name="s")

    assert nfull % 2 == 0

    @functools.partial(
        pl.kernel,
        mesh=mesh,
        out_type=jax.ShapeDtypeStruct((NC, N, D), jnp.float32),
        scratch_types=[
            pltpu.VMEM((B,), jnp.int32),
            pltpu.VMEM((B,), jnp.int32),
            pltpu.VMEM((B,), jnp.int32),
            pltpu.VMEM((B,), jnp.int32),
            pltpu.VMEM((B, D), jnp.float32),
            pltpu.VMEM((B, D), jnp.float32),
            pltpu.VMEM((B, D), jnp.float32),
            pltpu.VMEM_SHARED((N, D), jnp.float32),
            pltpu.SemaphoreType.DMA,
            pltpu.SemaphoreType.DMA,
            pltpu.SemaphoreType.DMA,
            pltpu.SemaphoreType.DMA,
            pltpu.SemaphoreType.DMA,
        ],
    )
    def k(node_hbm, src_hbm, dst_hbm, edge_hbm, part_hbm,
          si0, si1, di0, di1, m0, m1, g, acc,
          is0, is1, es0, es1, gsem):
        src_idx, dst_idx = (si0, si1), (di0, di1)
        m = (m0, m1)
        isem, esem = (is0, is1), (es0, es1)
        cid = lax.axis_index("c")
        sid = lax.axis_index("s")
        wid = sid * NC + cid

        def zrow(r, _):
            for c in range(D // LANES):
                m0[r, pl.ds(c * LANES, LANES)] = jnp.zeros((LANES,), jnp.float32)
            return 0
        lax.fori_loop(0, B, zrow, 0)
        for kk in range(rounds):
            j = kk * NS + sid
            @pl.when(j < nch)
            def _():
                pltpu.sync_copy(m0.at[pl.ds(0, ch)], acc.at[pl.ds(j * ch, ch)])
        plsc.subcore_barrier()

        def issue(blk, b):
            off = blk * B
            pltpu.async_copy(src_hbm.at[pl.ds(off, B)], src_idx[b], isem[b])
            pltpu.async_copy(dst_hbm.at[pl.ds(off, B)], dst_idx[b], isem[b])
            pltpu.async_copy(edge_hbm.at[pl.ds(off, B)], m[b], esem[b])

        def wait_idx(b):
            pltpu.make_async_copy(src_hbm.at[pl.ds(0, B)], src_idx[b],
                                  isem[b]).wait()
            pltpu.make_async_copy(dst_hbm.at[pl.ds(0, B)], dst_idx[b],
                                  isem[b]).wait()

        def wait_edge(b):
            pltpu.make_async_copy(edge_hbm.at[pl.ds(0, B)], m[b], esem[b]).wait()

        def compute(b):
            mb = m[b]

            def row(r, _):
                for rr in range(2):
                    for c in range(D // LANES):
                        sl = pl.ds(c * LANES, LANES)
                        r2 = r * 2 + rr
                        mb[r2, sl] = jnp.maximum(mb[r2, sl] + g[r2, sl], 0.0)
                return 0
            lax.fori_loop(0, B // 2, row, 0)

        def scatter(b):
            pltpu.sync_copy(m[b], acc.at[dst_idx[b]], add=True)

        first = wid * nfull
        issue(first, 0)

        def body(i2, _):
            for b in (0, 1):
                i = i2 * 2 + b
                blk = first + i
                q = 1 - b
                wait_idx(b)
                gd = pltpu.async_copy(node_hbm.at[src_idx[b]], g, gsem)
                @pl.when(i >= 1)
                def _():
                    scatter(q)
                @pl.when(i + 1 < nfull)
                def _():
                    issue(blk + 1, q)
                wait_edge(b)
                gd.wait()
                compute(b)
            return 0
        lax.fori_loop(0, nfull // 2, body, 0)
        scatter(1)

        if nextra:
            @pl.when(wid < nextra)
            def _():
                blk = nworkers * nfull + wid
                issue(blk, 0)
                wait_idx(0)
                wait_edge(0)
                pltpu.async_copy(node_hbm.at[si0], g, gsem).wait()
                compute(0)
                scatter(0)

        plsc.subcore_barrier()
        for kk in range(rounds):
            j = kk * NS + sid
            @pl.when(j < nch)
            def _():
                r0 = j * ch
                pltpu.sync_copy(acc.at[pl.ds(r0, ch)], m0.at[pl.ds(0, ch)])
                pltpu.sync_copy(m0.at[pl.ds(0, ch)],
                                part_hbm.at[cid, pl.ds(r0, ch)])

    return k


def _combine(x_ref, p_ref, o_ref):
    o_ref[...] = x_ref[...] + p_ref[0] + p_ref[1]


def kernel(node_feat, edge_index, edge_feat):
    N, D = node_feat.shape
    E = edge_feat.shape[0]
    src = edge_index[0]
    dst = edge_index[1]
    parts = _sc_message_pass(N, D, E)(node_feat, src, dst, edge_feat)

    rb = 1000 if N % 1000 == 0 else N
    out = pl.pallas_call(
        _combine,
        grid=(N // rb,),
        in_specs=[
            pl.BlockSpec((rb, D), lambda i: (i, 0)),
            pl.BlockSpec((NC, rb, D), lambda i: (0, i, 0)),
        ],
        out_specs=pl.BlockSpec((rb, D), lambda i: (i, 0)),
        out_shape=jax.ShapeDtypeStruct((N, D), jnp.float32),
    )(node_feat, parts)
    return out
